# baseline (speedup 1.0000x reference)
import jax
import jax.numpy as jnp
from jax.experimental import pallas as pl
from jax.experimental.pallas import tpu as pltpu

_STATE_DIM = 128
_SEM_DIM = 64
_SLOTS = 8192
_LR = 0.01
_N = 4096
_CHUNK = 128
_SEG = 1024
_NSEG = _SLOTS // _SEG
_TBLK = 1024
_INF = float('inf')


def _consolidate_kernel(states_ref, rewards_ref, w_ref, b_ref,
                        traces_ref, num_ref, ms_ref,
                        sem_ref, strengths_ref, tt_ref, dt_ref,
                        norms_ref, mrows_ref, mids_ref, mpen_ref):
    sem_ref[...] = jax.lax.dot_general(
        states_ref[...], w_ref[...],
        dimension_numbers=(((1,), (1,)), ((), ())),
        preferred_element_type=jnp.float32) + b_ref[...]

    traces_ref[...] = jnp.zeros((_SLOTS, _SEM_DIM), jnp.float32)
    strengths_ref[...] = jnp.zeros((_SLOTS, 1), jnp.float32)

    row_ids = jax.lax.broadcasted_iota(jnp.int32, (_SLOTS, 1), 0)
    seg_lane_ids = jax.lax.broadcasted_iota(jnp.int32, (1, _SEG), 1)
    mrow_ids = jax.lax.broadcasted_iota(jnp.int32, (_CHUNK, 1), 0)

    def chunk_body(c, num0):
        cs = c * _CHUNK
        for tb in range(_SLOTS // _TBLK):
            blk = traces_ref[tb * _TBLK:(tb + 1) * _TBLK, :]
            tt_ref[:, tb * _TBLK:(tb + 1) * _TBLK] = blk.T
        cblk = sem_ref[pl.ds(cs, _CHUNK), :]
        for sg in range(_NSEG):
            ttseg = tt_ref[:, sg * _SEG:(sg + 1) * _SEG]
            dt_ref[:, sg, :] = jax.lax.dot_general(
                cblk, ttseg,
                dimension_numbers=(((1,), (0,)), ((), ())),
                preferred_element_type=jnp.float32)
            norms_ref[sg:sg + 1, :] = jnp.sum(
                ttseg * ttseg, axis=0, keepdims=True)
        mids_ref[...] = jnp.full((_CHUNK, 1), -1, jnp.int32)
        mpen_ref[...] = jnp.zeros((_CHUNK, 1), jnp.float32)

        def step(k, num):
            i = cs + k
            content = sem_ref[pl.ds(i, 1), :]
            cnorm = jnp.sum(content * content)

            def seg_scan(sg, scarry):
                smin, sj = scarry
                d2p = (norms_ref[pl.ds(sg, 1), :]
                       - 2.0 * dt_ref[k, pl.ds(sg, 1), :])
                ids = sg * _SEG + seg_lane_ids
                d2m = jnp.where(ids < num0, d2p, _INF)
                gmin = jnp.min(d2m)
                gj = jnp.min(jnp.where(d2m == gmin, ids, _SLOTS))
                take = gmin < smin
                return (jnp.where(take, gmin, smin),
                        jnp.where(take, gj, sj))

            nseg = (num0 + (_SEG - 1)) // _SEG
            bmin_p, bj = jax.lax.fori_loop(
                0, nseg, seg_scan, (jnp.float32(_INF), jnp.int32(0)))
            base_d2 = bmin_p + cnorm

            bdiffs = mrows_ref[...] - content
            bd2 = jnp.sum(bdiffs * bdiffs, axis=1, keepdims=True)
            bd2m = jnp.where(mrow_ids < k, bd2 + mpen_ref[...], _INF)
            bmin_b = jnp.min(bd2m)
            mj = jnp.min(jnp.where(bd2m == bmin_b, mrow_ids, _CHUNK))
            slot_b = jnp.max(jnp.where(mrow_ids == mj, mids_ref[...], -1))

            use_buf = bmin_b < base_d2
            dmin = jnp.where(use_buf, bmin_b, base_d2)
            j = jnp.where(use_buf, slot_b, bj)
            do_update = (num > 0) & (dmin < 4.0)

            reward = jnp.abs(rewards_ref[pl.ds(i, 1), :][0, 0])
            eff_lr = _LR * (1.0 + reward)
            old = traces_ref[pl.ds(j, 1), :]
            upd = old + (content - old) * eff_lr
            s_old = strengths_ref[pl.ds(j, 1), :]

            tgt = jnp.where(do_update, j, num)
            newrow = jnp.where(do_update, upd, content)
            traces_ref[pl.ds(tgt, 1), :] = newrow
            strengths_ref[pl.ds(tgt, 1), :] = jnp.where(
                do_update, s_old + 1.0, 1.0)

            seg_t = tgt // _SEG
            lane_t = tgt - seg_t * _SEG
            nrow = norms_ref[pl.ds(seg_t, 1), :]
            norms_ref[pl.ds(seg_t, 1), :] = jnp.where(
                seg_lane_ids == lane_t, _INF, nrow)
            mpen_ref[...] = jnp.where(
                mids_ref[...] == tgt, _INF, mpen_ref[...])
            mrows_ref[pl.ds(k, 1), :] = newrow
            mids_ref[pl.ds(k, 1), :] = jnp.full((1, 1), tgt, jnp.int32)
            mpen_ref[pl.ds(k, 1), :] = jnp.zeros((1, 1), jnp.float32)
            return jnp.where(do_update, num, num + 1)

        return jax.lax.fori_loop(0, _CHUNK, step, num0)

    num = jax.lax.fori_loop(0, _N // _CHUNK, chunk_body, jnp.int32(0))

    valid = row_ids < num
    total = jnp.sum(jnp.where(valid, strengths_ref[...], 0.0))
    ms = jnp.where(num > 0, total / jnp.maximum(num, 1).astype(jnp.float32), 0.0)
    num_ref[...] = jnp.full((1, 1), num, jnp.int32)
    ms_ref[...] = jnp.full((1, 1), ms, jnp.float32)


@jax.jit
def kernel(replayed_states, replayed_rewards, W, b):
    rewards2 = replayed_rewards.reshape(_N, 1)
    b2 = b.reshape(1, _SEM_DIM)
    traces, num, ms = pl.pallas_call(
        _consolidate_kernel,
        out_shape=[
            jax.ShapeDtypeStruct((_SLOTS, _SEM_DIM), jnp.float32),
            jax.ShapeDtypeStruct((1, 1), jnp.int32),
            jax.ShapeDtypeStruct((1, 1), jnp.float32),
        ],
        scratch_shapes=[
            pltpu.VMEM((_N, _SEM_DIM), jnp.float32),
            pltpu.VMEM((_SLOTS, 1), jnp.float32),
            pltpu.VMEM((_SEM_DIM, _SLOTS), jnp.float32),
            pltpu.VMEM((_CHUNK, _NSEG, _SEG), jnp.float32),
            pltpu.VMEM((_NSEG, _SEG), jnp.float32),
            pltpu.VMEM((_CHUNK, _SEM_DIM), jnp.float32),
            pltpu.VMEM((_CHUNK, 1), jnp.int32),
            pltpu.VMEM((_CHUNK, 1), jnp.float32),
        ],
    )(replayed_states, rewards2, W, b2)
    return (jnp.array(True), jnp.array(_N, jnp.int32), num[0, 0], ms[0, 0],
            traces)


# speedup vs baseline: 1.3549x; 1.3549x over previous
import jax
import jax.numpy as jnp
from jax.experimental import pallas as pl
from jax.experimental.pallas import tpu as pltpu

_STATE_DIM = 128
_SEM_DIM = 64
_SLOTS = 8192
_LR = 0.01
_N = 4096
_LANE = 128
_MAC = 8
_MACSZ = _LANE * _MAC
_INF = float('inf')


def _consolidate_kernel(states_ref, rewards_ref, w_ref, b_ref,
                        traces_ref, num_ref, ms_ref,
                        sem_ref, strengths_ref, tt_ref):
    sem_ref[...] = jax.lax.dot_general(
        states_ref[...], w_ref[...],
        dimension_numbers=(((1,), (1,)), ((), ())),
        preferred_element_type=jnp.float32) + b_ref[...]

    traces_ref[...] = jnp.zeros((_SLOTS, _SEM_DIM), jnp.float32)
    strengths_ref[...] = jnp.zeros((_SLOTS, 1), jnp.float32)
    tt_ref[...] = jnp.zeros((_SLOTS // _LANE, _SEM_DIM, _LANE), jnp.float32)

    row_ids = jax.lax.broadcasted_iota(jnp.int32, (_SLOTS, 1), 0)
    seg_ids = jax.lax.broadcasted_iota(jnp.int32, (_MAC, _LANE), 0)
    lane_ids = jax.lax.broadcasted_iota(jnp.int32, (_MAC, _LANE), 1)
    lane_ids3 = jax.lax.broadcasted_iota(jnp.int32, (1, 1, _LANE), 2)

    def step(i, carry):
        num = carry
        content = sem_ref[pl.ds(i, 1), :]
        ccol = content.reshape(_SEM_DIM, 1)
        ccol3 = ccol.reshape(1, _SEM_DIM, 1)

        def mac_scan(m, dcarry):
            dmin, jmin = dcarry
            tiles = tt_ref[pl.ds(m * _MAC, _MAC), :, :]
            diffs = tiles - ccol3
            d2 = jnp.sum(diffs * diffs, axis=1)
            ids = m * _MACSZ + seg_ids * _LANE + lane_ids
            d2m = jnp.where(ids < num, d2, _INF)
            bmin = jnp.min(d2m)
            bj = jnp.min(jnp.where(d2m == bmin, ids, _SLOTS))
            take = bmin < dmin
            return (jnp.where(take, bmin, dmin),
                    jnp.where(take, bj, jmin))

        nmac = (num + (_MACSZ - 1)) // _MACSZ
        dmin, j = jax.lax.fori_loop(
            0, nmac, mac_scan, (jnp.float32(_INF), jnp.int32(0)))
        do_update = (num > 0) & (dmin < 4.0)

        reward = jnp.abs(rewards_ref[pl.ds(i, 1), :][0, 0])
        eff_lr = _LR * (1.0 + reward)
        old = traces_ref[pl.ds(j, 1), :]
        upd = old + (content - old) * eff_lr
        s_old = strengths_ref[pl.ds(j, 1), :]

        tgt = jnp.where(do_update, j, num)
        newrow = jnp.where(do_update, upd, content)
        traces_ref[pl.ds(tgt, 1), :] = newrow
        strengths_ref[pl.ds(tgt, 1), :] = jnp.where(do_update, s_old + 1.0, 1.0)

        sg = tgt // _LANE
        lane_t = tgt - sg * _LANE
        tile = tt_ref[pl.ds(sg, 1), :, :]
        newcol3 = newrow.reshape(1, _SEM_DIM, 1)
        tt_ref[pl.ds(sg, 1), :, :] = jnp.where(
            lane_ids3 == lane_t, newcol3, tile)
        return jnp.where(do_update, num, num + 1)

    num = jax.lax.fori_loop(0, _N, step, jnp.int32(0))

    valid = row_ids < num
    total = jnp.sum(jnp.where(valid, strengths_ref[...], 0.0))
    ms = jnp.where(num > 0, total / jnp.maximum(num, 1).astype(jnp.float32), 0.0)
    num_ref[...] = jnp.full((1, 1), num, jnp.int32)
    ms_ref[...] = jnp.full((1, 1), ms, jnp.float32)


@jax.jit
def kernel(replayed_states, replayed_rewards, W, b):
    rewards2 = replayed_rewards.reshape(_N, 1)
    b2 = b.reshape(1, _SEM_DIM)
    traces, num, ms = pl.pallas_call(
        _consolidate_kernel,
        out_shape=[
            jax.ShapeDtypeStruct((_SLOTS, _SEM_DIM), jnp.float32),
            jax.ShapeDtypeStruct((1, 1), jnp.int32),
            jax.ShapeDtypeStruct((1, 1), jnp.float32),
        ],
        scratch_shapes=[
            pltpu.VMEM((_N, _SEM_DIM), jnp.float32),
            pltpu.VMEM((_SLOTS, 1), jnp.float32),
            pltpu.VMEM((_SLOTS // _LANE, _SEM_DIM, _LANE), jnp.float32),
        ],
    )(replayed_states, rewards2, W, b2)
    return (jnp.array(True), jnp.array(_N, jnp.int32), num[0, 0], ms[0, 0],
            traces)


# speedup vs baseline: 1.8842x; 1.3906x over previous
import jax
import jax.numpy as jnp
from jax.experimental import pallas as pl
from jax.experimental.pallas import tpu as pltpu

_STATE_DIM = 128
_SEM_DIM = 64
_SLOTS = 8192
_LR = 0.01
_N = 4096
_LANE = 128
_MAC = 16
_MACSZ = _LANE * _MAC
_INF = float('inf')


def _consolidate_kernel(states_ref, rewards_ref, w_ref, b_ref,
                        traces_ref, num_ref, ms_ref,
                        sem_ref, strengths_ref, tt_ref):
    sem_ref[...] = jax.lax.dot_general(
        states_ref[...], w_ref[...],
        dimension_numbers=(((1,), (1,)), ((), ())),
        preferred_element_type=jnp.float32) + b_ref[...]

    traces_ref[...] = jnp.zeros((_SLOTS, _SEM_DIM), jnp.float32)
    strengths_ref[...] = jnp.zeros((_SLOTS, 1), jnp.float32)
    tt_ref[...] = jnp.zeros((_SLOTS // _LANE, _SEM_DIM, _LANE), jnp.float32)

    row_ids = jax.lax.broadcasted_iota(jnp.int32, (_SLOTS, 1), 0)
    seg_ids = jax.lax.broadcasted_iota(jnp.int32, (_MAC, _LANE), 0)
    lane_ids = jax.lax.broadcasted_iota(jnp.int32, (_MAC, _LANE), 1)
    lane_ids3 = jax.lax.broadcasted_iota(jnp.int32, (1, 1, _LANE), 2)

    def step(i, carry):
        num = carry
        content = sem_ref[pl.ds(i, 1), :]
        ccol = content.reshape(_SEM_DIM, 1)
        ccol3 = ccol.reshape(1, _SEM_DIM, 1)

        def mac_scan(m, dcarry):
            dmin, jmin = dcarry
            tiles = tt_ref[pl.ds(m * _MAC, _MAC), :, :]
            diffs = tiles - ccol3
            d2 = jnp.sum(diffs * diffs, axis=1)
            ids = m * _MACSZ + seg_ids * _LANE + lane_ids
            d2m = jnp.where(ids < num, d2, _INF)
            bmin = jnp.min(d2m)
            bj = jnp.min(jnp.where(d2m == bmin, ids, _SLOTS))
            take = bmin < dmin
            return (jnp.where(take, bmin, dmin),
                    jnp.where(take, bj, jmin))

        nmac = (num + (_MACSZ - 1)) // _MACSZ
        dmin, j = jax.lax.fori_loop(
            0, nmac, mac_scan, (jnp.float32(_INF), jnp.int32(0)))
        do_update = (num > 0) & (dmin < 4.0)

        reward = jnp.abs(rewards_ref[pl.ds(i, 1), :][0, 0])
        eff_lr = _LR * (1.0 + reward)
        old = traces_ref[pl.ds(j, 1), :]
        upd = old + (content - old) * eff_lr
        s_old = strengths_ref[pl.ds(j, 1), :]

        tgt = jnp.where(do_update, j, num)
        newrow = jnp.where(do_update, upd, content)
        traces_ref[pl.ds(tgt, 1), :] = newrow
        strengths_ref[pl.ds(tgt, 1), :] = jnp.where(do_update, s_old + 1.0, 1.0)

        sg = tgt // _LANE
        lane_t = tgt - sg * _LANE
        tile = tt_ref[pl.ds(sg, 1), :, :]
        newcol3 = newrow.reshape(1, _SEM_DIM, 1)
        tt_ref[pl.ds(sg, 1), :, :] = jnp.where(
            lane_ids3 == lane_t, newcol3, tile)
        return jnp.where(do_update, num, num + 1)

    num = jax.lax.fori_loop(0, _N, step, jnp.int32(0))

    valid = row_ids < num
    total = jnp.sum(jnp.where(valid, strengths_ref[...], 0.0))
    ms = jnp.where(num > 0, total / jnp.maximum(num, 1).astype(jnp.float32), 0.0)
    num_ref[...] = jnp.full((1, 1), num, jnp.int32)
    ms_ref[...] = jnp.full((1, 1), ms, jnp.float32)


@jax.jit
def kernel(replayed_states, replayed_rewards, W, b):
    rewards2 = replayed_rewards.reshape(_N, 1)
    b2 = b.reshape(1, _SEM_DIM)
    traces, num, ms = pl.pallas_call(
        _consolidate_kernel,
        out_shape=[
            jax.ShapeDtypeStruct((_SLOTS, _SEM_DIM), jnp.float32),
            jax.ShapeDtypeStruct((1, 1), jnp.int32),
            jax.ShapeDtypeStruct((1, 1), jnp.float32),
        ],
        scratch_shapes=[
            pltpu.VMEM((_N, _SEM_DIM), jnp.float32),
            pltpu.VMEM((_SLOTS, 1), jnp.float32),
            pltpu.VMEM((_SLOTS // _LANE, _SEM_DIM, _LANE), jnp.float32),
        ],
    )(replayed_states, rewards2, W, b2)
    return (jnp.array(True), jnp.array(_N, jnp.int32), num[0, 0], ms[0, 0],
            traces)
